# baseline (speedup 1.0000x reference)
import jax
import jax.numpy as jnp
from jax import lax
from jax.experimental import pallas as pl
from jax.experimental.pallas import tpu as pltpu
from jax.experimental.pallas import tpu_sc as plsc

K = 64
DIM = 64
NC = 2
NS = 16
L = 16
NW = NC * NS
N = 131072
NH = 40960
CHUNK = NH // NW
WIN = CHUNK
NB = DIM // 8
CS = DIM + 3
ROW = L * CS
TB = 2048
NTB = (N - NH) // TB


def _sc_body(xt_hbm, ids_hbm, out, band, ids, acc, sem_ids):
    cid = lax.axis_index("c")
    sid = lax.axis_index("s")
    wid = cid * NS + sid

    zero = jnp.zeros((L,), jnp.float32)
    ones = jnp.full((L,), 1.0, jnp.float32)
    laneoff = lax.iota(jnp.int32, L) * jnp.int32(CS)

    d_ids = pltpu.async_copy(
        ids_hbm.at[pl.ds(pl.multiple_of(wid * CHUNK, CHUNK), CHUNK)], ids,
        sem_ids)

    def _zrow(r, _):
        for j in range(ROW // L):
            acc[r, pl.ds(j * L, L)] = zero
        return 0

    lax.fori_loop(0, K, _zrow, 0)
    d_ids.wait()

    nbase = pl.multiple_of(wid * CHUNK, WIN)
    for b in range(NB):
        pltpu.sync_copy(xt_hbm.at[pl.ds(b * 8, 8), pl.ds(nbase, WIN)], band)

        @plsc.parallel_loop(0, WIN // L, 1, unroll=2)
        def _g(g):
            cvec = ids[pl.ds(g * L, L)]
            col = laneoff
            v = band[0, pl.ds(g * L, L)]
            plsc.addupdate_scatter(acc, [cvec, col + jnp.int32(b * 8)], v)
            qv = v * v
            for r in range(1, 8):
                v = band[r, pl.ds(g * L, L)]
                plsc.addupdate_scatter(
                    acc, [cvec, col + jnp.int32(b * 8 + r)], v)
                qv = qv + v * v
            plsc.addupdate_scatter(acc, [cvec, col + jnp.int32(DIM)], qv)
            if b == 0:
                plsc.addupdate_scatter(
                    acc, [cvec, col + jnp.int32(DIM + 1)], ones)

    pltpu.sync_copy(acc, out.at[pl.ds(pl.multiple_of(wid * K, K), K)])


def _sc_moments(xt, clustering):
    mesh = plsc.VectorSubcoreMesh(core_axis_name="c", subcore_axis_name="s",
                                  num_cores=NC, num_subcores=NS)
    f = pl.kernel(
        _sc_body,
        out_type=[jax.ShapeDtypeStruct((NW * K, ROW), jnp.float32)],
        mesh=mesh,
        scratch_types=[
            pltpu.VMEM((8, WIN), jnp.float32),
            pltpu.VMEM((CHUNK,), jnp.int32),
            pltpu.VMEM((K, ROW), jnp.float32),
            pltpu.SemaphoreType.DMA,
        ],
        compiler_params=pltpu.CompilerParams(use_tc_tiling_on_sc=True,
                                             needs_layout_passes=False),
    )
    return f(xt, clustering)


def _tc_moments_body(x_ref, c_ref, o_ref):
    i = pl.program_id(0)

    @pl.when(i == 0)
    def _():
        o_ref[...] = jnp.zeros((K, CS), jnp.float32)

    x = x_ref[...]
    c = c_ref[...]
    oh = (c[None, :] == lax.broadcasted_iota(jnp.int32, (K, TB), 0)
          ).astype(jnp.float32)
    q = jnp.sum(x * x, axis=0, keepdims=True)
    q2 = jnp.concatenate(
        [q, jnp.ones((1, TB), jnp.float32),
         jnp.zeros((CS - DIM - 2, TB), jnp.float32)], axis=0)
    o_ref[:, :DIM] += lax.dot_general(
        oh, x, (((1,), (1,)), ((), ())),
        preferred_element_type=jnp.float32,
        precision=lax.Precision.DEFAULT)
    o_ref[:, DIM:] += lax.dot_general(
        oh, q2, (((1,), (1,)), ((), ())),
        preferred_element_type=jnp.float32,
        precision=lax.Precision.DEFAULT)


def _tc_moments(xt, clustering):
    return pl.pallas_call(
        _tc_moments_body,
        grid=(NTB,),
        in_specs=[
            pl.BlockSpec((DIM, TB), lambda i: (0, NH // TB + i)),
            pl.BlockSpec((TB,), lambda i: (NH // TB + i,)),
        ],
        out_specs=pl.BlockSpec((K, CS), lambda i: (0, 0)),
        out_shape=jax.ShapeDtypeStruct((K, CS), jnp.float32),
    )(xt, clustering)


def _finalize_body(p_ref, t_ref, o_ref):
    s = jnp.sum(p_ref[...], axis=0)
    a66 = s[:, 0:CS]
    for l in range(1, L):
        a66 = a66 + s[:, l * CS:(l + 1) * CS]
    a66 = a66 + t_ref[...]
    sx = a66[:, :DIM]
    q = a66[:, DIM:DIM + 1]
    m = a66[:, DIM + 1:DIM + 2]
    cnt = m + 1.0
    ai = (sx + 0.001) / cnt
    si_sum = (0.001 + q
              - 2.0 * jnp.sum(ai * sx, axis=1, keepdims=True)
              + m * jnp.sum(ai * ai, axis=1, keepdims=True))
    si = jnp.sqrt(si_sum / cnt)

    diff = ai[:, None, :] - ai[None, :, :]
    mij = jnp.sqrt(jnp.sum(diff * diff, axis=-1))
    ones = jnp.ones((K, 1), jnp.float32)
    si_j = lax.dot_general(ones, si, (((1,), (1,)), ((), ())),
                           preferred_element_type=jnp.float32)
    rsum = si + si_j
    safe_m = jnp.where(mij == 0.0, 1.0, mij)
    rij = jnp.where(mij == 0.0, 0.1, rsum / safe_m)
    ii = lax.broadcasted_iota(jnp.int32, (K, K), 0)
    jj = lax.broadcasted_iota(jnp.int32, (K, K), 1)
    rij = jnp.where(ii == jj, 0.0, rij)
    di = jnp.max(rij, axis=1, keepdims=True)
    o_ref[...] = jnp.sum(di, axis=0, keepdims=True) / jnp.float32(K)


def _finalize(partials, tc_part):
    return pl.pallas_call(
        _finalize_body,
        out_shape=jax.ShapeDtypeStruct((1, 1), jnp.float32),
    )(partials, tc_part)


@jax.jit
def kernel(data_points, clustering):
    xt = data_points.T
    (partials,) = _sc_moments(xt, clustering)
    tc_part = _tc_moments(xt, clustering)
    out = _finalize(partials.reshape(NW, K, ROW), tc_part)
    return out[0, 0]


# speedup vs baseline: 1.0399x; 1.0399x over previous
import jax
import jax.numpy as jnp
from jax import lax
from jax.experimental import pallas as pl
from jax.experimental.pallas import tpu as pltpu
from jax.experimental.pallas import tpu_sc as plsc

K = 64
DIM = 64
NC = 2
NS = 16
L = 16
NW = NC * NS
N = 131072
NH = 49152
CHUNK = NH // NW
WIN = CHUNK
NB = DIM // 8
CS = DIM + 3
ROW = L * CS
TB = 2048
NTB = (N - NH) // TB


def _sc_body(xt_hbm, ids_hbm, out, band, ids, acc, sem_ids):
    cid = lax.axis_index("c")
    sid = lax.axis_index("s")
    wid = cid * NS + sid

    zero = jnp.zeros((L,), jnp.float32)
    ones = jnp.full((L,), 1.0, jnp.float32)
    laneoff = lax.iota(jnp.int32, L) * jnp.int32(CS)

    d_ids = pltpu.async_copy(
        ids_hbm.at[pl.ds(pl.multiple_of(wid * CHUNK, CHUNK), CHUNK)], ids,
        sem_ids)

    def _zrow(r, _):
        for j in range(ROW // L):
            acc[r, pl.ds(j * L, L)] = zero
        return 0

    lax.fori_loop(0, K, _zrow, 0)
    d_ids.wait()

    nbase = pl.multiple_of(wid * CHUNK, WIN)
    for b in range(NB):
        pltpu.sync_copy(xt_hbm.at[pl.ds(b * 8, 8), pl.ds(nbase, WIN)], band)

        @plsc.parallel_loop(0, WIN // L, 1, unroll=2)
        def _g(g):
            cvec = ids[pl.ds(g * L, L)]
            col = laneoff
            v = band[0, pl.ds(g * L, L)]
            plsc.addupdate_scatter(acc, [cvec, col + jnp.int32(b * 8)], v)
            qv = v * v
            for r in range(1, 8):
                v = band[r, pl.ds(g * L, L)]
                plsc.addupdate_scatter(
                    acc, [cvec, col + jnp.int32(b * 8 + r)], v)
                qv = qv + v * v
            plsc.addupdate_scatter(acc, [cvec, col + jnp.int32(DIM)], qv)
            if b == 0:
                plsc.addupdate_scatter(
                    acc, [cvec, col + jnp.int32(DIM + 1)], ones)

    pltpu.sync_copy(acc, out.at[pl.ds(pl.multiple_of(wid * K, K), K)])


def _sc_moments(xt, clustering):
    mesh = plsc.VectorSubcoreMesh(core_axis_name="c", subcore_axis_name="s",
                                  num_cores=NC, num_subcores=NS)
    f = pl.kernel(
        _sc_body,
        out_type=[jax.ShapeDtypeStruct((NW * K, ROW), jnp.float32)],
        mesh=mesh,
        scratch_types=[
            pltpu.VMEM((8, WIN), jnp.float32),
            pltpu.VMEM((CHUNK,), jnp.int32),
            pltpu.VMEM((K, ROW), jnp.float32),
            pltpu.SemaphoreType.DMA,
        ],
        compiler_params=pltpu.CompilerParams(use_tc_tiling_on_sc=True,
                                             needs_layout_passes=False),
    )
    return f(xt, clustering)


def _tc_moments_body(x_ref, c_ref, o_ref):
    i = pl.program_id(0)

    @pl.when(i == 0)
    def _():
        o_ref[...] = jnp.zeros((K, CS), jnp.float32)

    x = x_ref[...]
    c = c_ref[...]
    oh = (c[None, :] == lax.broadcasted_iota(jnp.int32, (K, TB), 0)
          ).astype(jnp.float32)
    q = jnp.sum(x * x, axis=0, keepdims=True)
    q2 = jnp.concatenate(
        [q, jnp.ones((1, TB), jnp.float32),
         jnp.zeros((CS - DIM - 2, TB), jnp.float32)], axis=0)
    o_ref[:, :DIM] += lax.dot_general(
        oh, x, (((1,), (1,)), ((), ())),
        preferred_element_type=jnp.float32,
        precision=lax.Precision.DEFAULT)
    o_ref[:, DIM:] += lax.dot_general(
        oh, q2, (((1,), (1,)), ((), ())),
        preferred_element_type=jnp.float32,
        precision=lax.Precision.DEFAULT)


def _tc_moments(xt, clustering):
    return pl.pallas_call(
        _tc_moments_body,
        grid=(NTB,),
        in_specs=[
            pl.BlockSpec((DIM, TB), lambda i: (0, NH // TB + i)),
            pl.BlockSpec((TB,), lambda i: (NH // TB + i,)),
        ],
        out_specs=pl.BlockSpec((K, CS), lambda i: (0, 0)),
        out_shape=jax.ShapeDtypeStruct((K, CS), jnp.float32),
    )(xt, clustering)


def _finalize_body(p_ref, t_ref, o_ref):
    s = jnp.sum(p_ref[...], axis=0)
    a66 = s[:, 0:CS]
    for l in range(1, L):
        a66 = a66 + s[:, l * CS:(l + 1) * CS]
    a66 = a66 + t_ref[...]
    sx = a66[:, :DIM]
    q = a66[:, DIM:DIM + 1]
    m = a66[:, DIM + 1:DIM + 2]
    cnt = m + 1.0
    ai = (sx + 0.001) / cnt
    si_sum = (0.001 + q
              - 2.0 * jnp.sum(ai * sx, axis=1, keepdims=True)
              + m * jnp.sum(ai * ai, axis=1, keepdims=True))
    si = jnp.sqrt(si_sum / cnt)

    diff = ai[:, None, :] - ai[None, :, :]
    mij = jnp.sqrt(jnp.sum(diff * diff, axis=-1))
    ones = jnp.ones((K, 1), jnp.float32)
    si_j = lax.dot_general(ones, si, (((1,), (1,)), ((), ())),
                           preferred_element_type=jnp.float32)
    rsum = si + si_j
    safe_m = jnp.where(mij == 0.0, 1.0, mij)
    rij = jnp.where(mij == 0.0, 0.1, rsum / safe_m)
    ii = lax.broadcasted_iota(jnp.int32, (K, K), 0)
    jj = lax.broadcasted_iota(jnp.int32, (K, K), 1)
    rij = jnp.where(ii == jj, 0.0, rij)
    di = jnp.max(rij, axis=1, keepdims=True)
    o_ref[...] = jnp.sum(di, axis=0, keepdims=True) / jnp.float32(K)


def _finalize(partials, tc_part):
    return pl.pallas_call(
        _finalize_body,
        out_shape=jax.ShapeDtypeStruct((1, 1), jnp.float32),
    )(partials, tc_part)


@jax.jit
def kernel(data_points, clustering):
    xt = data_points.T
    (partials,) = _sc_moments(xt, clustering)
    tc_part = _tc_moments(xt, clustering)
    out = _finalize(partials.reshape(NW, K, ROW), tc_part)
    return out[0, 0]
